# baseline (speedup 1.0000x reference)
import functools

import jax
import jax.numpy as jnp
from jax import lax
from jax.experimental import pallas as pl
from jax.experimental.pallas import tpu as pltpu
from jax.experimental.pallas import tpu_sc as plsc

N_CONCEPTS = 16384
NNZ = 2621440
D = 64
B = 1024

NC = 2
NS = 16
NW = NC * NS
EPT = NNZ // NW
K = 128
NBC = 32
CB = NBC * K
NBLKS = EPT // CB
CPT = EPT // K
ROWS_PER_SUB = N_CONCEPTS // NS


def _sc_body(rows_h, cols_h, vals_h, w_h, out_h,
             colsb, rowsb, valb, gbuf, sbuf, agg_sp,
             gsem0, gsem1, gsem2, gsem3, ssem0, ssem1):
    c = lax.axis_index("c")
    s = lax.axis_index("s")
    wid = c * NS + s
    gsems = (gsem0, gsem1, gsem2, gsem3)
    ssems = (ssem0, ssem1)

    zero16 = jnp.zeros((16,), jnp.float32)

    def _z(e, _):
        for q in range(4):
            sbuf[0, e, pl.ds(q * 16, 16)] = zero16
        return 0

    lax.fori_loop(0, K, _z, 0)

    def _zcopy(j, _):
        pltpu.sync_copy(sbuf.at[0], agg_sp.at[pl.ds(s * ROWS_PER_SUB + j * K, K)])
        return 0

    lax.fori_loop(0, ROWS_PER_SUB // K, _zcopy, 0)
    plsc.subcore_barrier()

    chunk0 = wid * CPT

    def _scale(gs, ss, j):
        mask = jnp.full((16,), -65536, jnp.int32)

        @plsc.parallel_loop(0, 8)
        def _grp(i):
            vv = valb[j, pl.ds(i * 16, 16)]
            for l in range(16):
                v16 = jnp.full((16,), vv[l], jnp.float32)
                e = i * 16 + l
                for h in range(2):
                    x = gbuf[gs, e, pl.ds(h * 16, 16)]
                    a = lax.bitcast_convert_type(x << 16, jnp.float32)
                    b = lax.bitcast_convert_type(x & mask, jnp.float32)
                    sbuf[ss, e, pl.ds(h * 32, 16)] = a * v16
                    sbuf[ss, e, pl.ds(h * 32 + 16, 16)] = b * v16

    def _block(blk, _):
        row0 = chunk0 + blk * NBC
        pltpu.sync_copy(cols_h.at[pl.ds(row0, NBC)], colsb)
        pltpu.sync_copy(rows_h.at[pl.ds(row0, NBC)], rowsb)
        pltpu.sync_copy(vals_h.at[pl.ds(row0, NBC)], valb)
        for gs in range(4):
            pltpu.async_copy(w_h.at[colsb.at[gs]], gbuf.at[gs], gsems[gs])

        def _quad(t, _):
            for jc in range(4):
                j = t * 4 + jc
                gs, ss = jc, jc % 2
                pltpu.make_async_copy(w_h.at[colsb.at[j]], gbuf.at[gs],
                                      gsems[gs]).wait()

                @pl.when(j >= 2)
                def _(ss=ss):
                    pltpu.make_async_copy(sbuf.at[ss], agg_sp.at[rowsb.at[0]],
                                          ssems[ss]).wait()

                _scale(gs, ss, j)

                @pl.when(j + 4 < NBC)
                def _(j=j, gs=gs):
                    pltpu.async_copy(w_h.at[colsb.at[j + 4]], gbuf.at[gs],
                                     gsems[gs])

                pltpu.async_copy(sbuf.at[ss], agg_sp.at[rowsb.at[j]],
                                 ssems[ss], add=True)
            return 0

        lax.fori_loop(0, NBC // 4, _quad, 0)
        for ss in range(2):
            pltpu.make_async_copy(sbuf.at[ss], agg_sp.at[rowsb.at[0]],
                                  ssems[ss]).wait()
        return 0

    lax.fori_loop(0, NBLKS, _block, 0)
    plsc.subcore_barrier()

    def _out(j, _):
        r0 = s * ROWS_PER_SUB + j * K
        pltpu.sync_copy(agg_sp.at[pl.ds(r0, K)], sbuf.at[0])
        pltpu.sync_copy(sbuf.at[0], out_h.at[c, pl.ds(r0, K)])
        return 0

    lax.fori_loop(0, ROWS_PER_SUB // K, _out, 0)


def _sc_spmm(rows2, cols2, vals2, w):
    mesh = plsc.VectorSubcoreMesh(core_axis_name="c", subcore_axis_name="s",
                                  num_cores=NC, num_subcores=NS)
    return pl.kernel(
        _sc_body,
        out_type=jax.ShapeDtypeStruct((NC, N_CONCEPTS, D), jnp.float32),
        mesh=mesh,
        scratch_types=[
            pltpu.VMEM((NBC, K), jnp.int32),
            pltpu.VMEM((NBC, K), jnp.int32),
            pltpu.VMEM((NBC, K), jnp.float32),
            pltpu.VMEM((4, K, D // 2), jnp.int32),
            pltpu.VMEM((2, K, D), jnp.float32),
            pltpu.VMEM_SHARED((N_CONCEPTS, D), jnp.float32),
            pltpu.SemaphoreType.DMA,
            pltpu.SemaphoreType.DMA,
            pltpu.SemaphoreType.DMA,
            pltpu.SemaphoreType.DMA,
            pltpu.SemaphoreType.DMA,
            pltpu.SemaphoreType.DMA,
        ],
        compiler_params=pltpu.CompilerParams(use_tc_tiling_on_sc=False),
    )(rows2, cols2, vals2, w)


NBLK = 1024


def _mm_body(x_ref, a0_ref, a1_ref, b_ref, o_ref):
    a = a0_ref[...] + a1_ref[...]
    o_ref[...] = lax.dot_general(
        x_ref[...], a, (((1,), (1,)), ((), ())),
        preferred_element_type=jnp.float32) + b_ref[...]


@functools.partial(jax.jit, donate_argnums=())
def _dense(inputs, agg2, b2):
    return pl.pallas_call(
        _mm_body,
        grid=(N_CONCEPTS // NBLK,),
        in_specs=[
            pl.BlockSpec((B, D), lambda n: (0, 0)),
            pl.BlockSpec((NBLK, D), lambda n: (n, 0)),
            pl.BlockSpec((NBLK, D), lambda n: (n, 0)),
            pl.BlockSpec((1, NBLK), lambda n: (0, n)),
        ],
        out_specs=pl.BlockSpec((B, NBLK), lambda n: (0, n)),
        out_shape=jax.ShapeDtypeStruct((B, N_CONCEPTS), jnp.float32),
    )(inputs, agg2[0], agg2[1], b2)


_PERM = sum(([h * 32 + p + 2 * k for k in range(16)]
             for h in range(2) for p in range(2)), [])


def kernel(inputs, rows, cols, vals, w, b):
    rows2 = rows.reshape(NNZ // K, K)
    cols2 = cols.reshape(NNZ // K, K)
    vals2 = vals.reshape(NNZ // K, K)
    wbits = lax.bitcast_convert_type(
        w.astype(jnp.bfloat16).reshape(N_CONCEPTS, D // 2, 2), jnp.int32)
    agg2 = _sc_spmm(rows2, cols2, vals2, wbits)
    inputs_p = inputs[:, jnp.array(_PERM, dtype=jnp.int32)]
    return _dense(inputs_p, agg2, b.reshape(1, N_CONCEPTS))


# speedup vs baseline: 1.4877x; 1.4877x over previous
import functools

import jax
import jax.numpy as jnp
from jax import lax
from jax.experimental import pallas as pl
from jax.experimental.pallas import tpu as pltpu
from jax.experimental.pallas import tpu_sc as plsc

N_CONCEPTS = 16384
NNZ = 2621440
D = 64
B = 1024

NC = 2
NS = 16
NW = NC * NS
EPT = NNZ // NW
K = 128
NBC = 32
CB = NBC * K
NBLKS = EPT // CB
CPT = EPT // K
ROWS_PER_SUB = N_CONCEPTS // NS


def _sc_body(rows_h, cols_h, vals_h, w_h, out_h,
             colsb, rowsb, valb, gbuf, sbuf, vexp, agg_sp,
             gsem0, gsem1, gsem2, gsem3, ssem0, ssem1):
    c = lax.axis_index("c")
    s = lax.axis_index("s")
    wid = c * NS + s
    gsems = (gsem0, gsem1, gsem2, gsem3)
    ssems = (ssem0, ssem1)

    zero16 = jnp.zeros((16,), jnp.float32)

    def _z(e, _):
        for q in range(4):
            sbuf[0, e, pl.ds(q * 16, 16)] = zero16
        return 0

    lax.fori_loop(0, K, _z, 0)

    def _zcopy(j, _):
        pltpu.sync_copy(sbuf.at[0], agg_sp.at[pl.ds(s * ROWS_PER_SUB + j * K, K)])
        return 0

    lax.fori_loop(0, ROWS_PER_SUB // K, _zcopy, 0)
    plsc.subcore_barrier()

    chunk0 = wid * CPT

    def _scale(gs, ss, j):
        mask = jnp.full((16,), -65536, jnp.int32)

        @plsc.parallel_loop(0, 8)
        def _prep(i):
            vv = valb[j, pl.ds(i * 16, 16)]
            for l in range(16):
                vexp[i * 16 + l, :] = jnp.full((16,), vv[l], jnp.float32)

        @plsc.parallel_loop(0, K)
        def _mul(e):
            v16 = vexp[e, :]
            for h in range(2):
                x = gbuf[gs, e, pl.ds(h * 16, 16)]
                a = lax.bitcast_convert_type(x << 16, jnp.float32)
                b = lax.bitcast_convert_type(x & mask, jnp.float32)
                sbuf[ss, e, pl.ds(h * 32, 16)] = a * v16
                sbuf[ss, e, pl.ds(h * 32 + 16, 16)] = b * v16

    def _block(blk, _):
        row0 = chunk0 + blk * NBC
        pltpu.sync_copy(cols_h.at[pl.ds(row0, NBC)], colsb)
        pltpu.sync_copy(rows_h.at[pl.ds(row0, NBC)], rowsb)
        pltpu.sync_copy(vals_h.at[pl.ds(row0, NBC)], valb)
        for gs in range(4):
            pltpu.async_copy(w_h.at[colsb.at[gs]], gbuf.at[gs], gsems[gs])

        def _quad(t, _):
            for jc in range(4):
                j = t * 4 + jc
                gs, ss = jc, jc % 2
                pltpu.make_async_copy(w_h.at[colsb.at[j]], gbuf.at[gs],
                                      gsems[gs]).wait()

                @pl.when(j >= 2)
                def _(ss=ss):
                    pltpu.make_async_copy(sbuf.at[ss], agg_sp.at[rowsb.at[0]],
                                          ssems[ss]).wait()

                _scale(gs, ss, j)

                @pl.when(j + 4 < NBC)
                def _(j=j, gs=gs):
                    pltpu.async_copy(w_h.at[colsb.at[j + 4]], gbuf.at[gs],
                                     gsems[gs])

                pltpu.async_copy(sbuf.at[ss], agg_sp.at[rowsb.at[j]],
                                 ssems[ss], add=True)
            return 0

        lax.fori_loop(0, NBC // 4, _quad, 0)
        for ss in range(2):
            pltpu.make_async_copy(sbuf.at[ss], agg_sp.at[rowsb.at[0]],
                                  ssems[ss]).wait()
        return 0

    lax.fori_loop(0, NBLKS, _block, 0)
    plsc.subcore_barrier()

    def _out(j, _):
        r0 = s * ROWS_PER_SUB + j * K
        pltpu.sync_copy(agg_sp.at[pl.ds(r0, K)], sbuf.at[0])
        pltpu.sync_copy(sbuf.at[0], out_h.at[c, pl.ds(r0, K)])
        return 0

    lax.fori_loop(0, ROWS_PER_SUB // K, _out, 0)


def _sc_spmm(rows2, cols2, vals2, w):
    mesh = plsc.VectorSubcoreMesh(core_axis_name="c", subcore_axis_name="s",
                                  num_cores=NC, num_subcores=NS)
    return pl.kernel(
        _sc_body,
        out_type=jax.ShapeDtypeStruct((NC, N_CONCEPTS, D), jnp.float32),
        mesh=mesh,
        scratch_types=[
            pltpu.VMEM((NBC, K), jnp.int32),
            pltpu.VMEM((NBC, K), jnp.int32),
            pltpu.VMEM((NBC, K), jnp.float32),
            pltpu.VMEM((4, K, D // 2), jnp.int32),
            pltpu.VMEM((2, K, D), jnp.float32),
            pltpu.VMEM((K, 16), jnp.float32),
            pltpu.VMEM_SHARED((N_CONCEPTS, D), jnp.float32),
            pltpu.SemaphoreType.DMA,
            pltpu.SemaphoreType.DMA,
            pltpu.SemaphoreType.DMA,
            pltpu.SemaphoreType.DMA,
            pltpu.SemaphoreType.DMA,
            pltpu.SemaphoreType.DMA,
        ],
        compiler_params=pltpu.CompilerParams(use_tc_tiling_on_sc=False),
    )(rows2, cols2, vals2, w)


NBLK = 1024


def _mm_body(x_ref, a0_ref, a1_ref, b_ref, o_ref):
    a = a0_ref[...] + a1_ref[...]
    o_ref[...] = lax.dot_general(
        x_ref[...], a, (((1,), (1,)), ((), ())),
        preferred_element_type=jnp.float32) + b_ref[...]


@functools.partial(jax.jit, donate_argnums=())
def _dense(inputs, agg2, b2):
    return pl.pallas_call(
        _mm_body,
        grid=(N_CONCEPTS // NBLK,),
        in_specs=[
            pl.BlockSpec((B, D), lambda n: (0, 0)),
            pl.BlockSpec((NBLK, D), lambda n: (n, 0)),
            pl.BlockSpec((NBLK, D), lambda n: (n, 0)),
            pl.BlockSpec((1, NBLK), lambda n: (0, n)),
        ],
        out_specs=pl.BlockSpec((B, NBLK), lambda n: (0, n)),
        out_shape=jax.ShapeDtypeStruct((B, N_CONCEPTS), jnp.float32),
    )(inputs, agg2[0], agg2[1], b2)


_PERM = sum(([h * 32 + p + 2 * k for k in range(16)]
             for h in range(2) for p in range(2)), [])


def kernel(inputs, rows, cols, vals, w, b):
    rows2 = rows.reshape(NNZ // K, K)
    cols2 = cols.reshape(NNZ // K, K)
    vals2 = vals.reshape(NNZ // K, K)
    wbits = lax.bitcast_convert_type(
        w.astype(jnp.bfloat16).reshape(N_CONCEPTS, D // 2, 2), jnp.int32)
    agg2 = _sc_spmm(rows2, cols2, vals2, wbits)
    inputs_p = inputs[:, jnp.array(_PERM, dtype=jnp.int32)]
    return _dense(inputs_p, agg2, b.reshape(1, N_CONCEPTS))
